# baseline (speedup 1.0000x reference)
import functools

import jax
import jax.numpy as jnp
from jax import lax
from jax.experimental import pallas as pl
from jax.experimental.pallas import tpu as pltpu
from jax.experimental.pallas import tpu_sc as plsc

N_NODES = 10000
N_EDGES = 320000
D = 128

NC = 2
NS = 16
NW = NC * NS
CH = 128
KCH = 80
HK = KCH // 2
EPW = KCH * CH
E_PAD = NW * EPW
HP = 10240
ROWS_PER_TILE = HP // NS


def _sc_segment_sum(x, src2d, dst2d):
    mesh = plsc.VectorSubcoreMesh(core_axis_name="c", subcore_axis_name="s")

    @functools.partial(
        pl.kernel,
        mesh=mesh,
        out_type=jax.ShapeDtypeStruct((NC, HP, D), jnp.float32),
        scratch_types=[
            pltpu.VMEM((HK, CH), jnp.int32),
            pltpu.VMEM((HK, CH), jnp.int32),
            pltpu.VMEM((2, CH, D), jnp.float32),
            pltpu.VMEM_SHARED((HP, D), jnp.float32),
            pltpu.SemaphoreType.DMA,
            pltpu.SemaphoreType.DMA,
        ],
    )
    def k(x_hbm, src_hbm, dst_hbm, out_hbm, src_v, dst_v, rows_v, h_sh,
          sem0, sem1):
        c = lax.axis_index("c")
        s = lax.axis_index("s")
        wid = s * NC + c

        ZB = 80
        def zrow(i, _):
            def zcol(j, _):
                rows_v[0, i, pl.ds(j * 16, 16)] = jnp.zeros((16,), jnp.float32)
                return 0
            return lax.fori_loop(0, D // 16, zcol, 0)
        lax.fori_loop(0, ZB, zrow, 0)

        base = s * ROWS_PER_TILE

        def zblk(t, _):
            pltpu.sync_copy(rows_v.at[0, pl.ds(0, ZB)],
                            h_sh.at[pl.ds(base + t * ZB, ZB)])
            return 0
        lax.fori_loop(0, ROWS_PER_TILE // ZB, zblk, 0)
        plsc.subcore_barrier()

        for half in range(2):
            pltpu.sync_copy(src_hbm.at[wid, pl.ds(half * HK, HK)], src_v)
            pltpu.sync_copy(dst_hbm.at[wid, pl.ds(half * HK, HK)], dst_v)

            def pair(t, _):
                j0 = 2 * t
                g0 = pltpu.async_copy(
                    x_hbm.at[src_v.at[j0]], rows_v.at[0], sem0)
                g1 = pltpu.async_copy(
                    x_hbm.at[src_v.at[j0 + 1]], rows_v.at[1], sem1)
                g0.wait()
                pltpu.sync_copy(rows_v.at[0], h_sh.at[dst_v.at[j0]], add=True)
                g1.wait()
                pltpu.sync_copy(
                    rows_v.at[1], h_sh.at[dst_v.at[j0 + 1]], add=True)
                return 0
            lax.fori_loop(0, HK // 2, pair, 0)
        plsc.subcore_barrier()

        pltpu.sync_copy(h_sh.at[pl.ds(base, ROWS_PER_TILE)],
                        out_hbm.at[c, pl.ds(base, ROWS_PER_TILE)])

    return k(x, src2d, dst2d)


def _tc_linear(h2, W, b2):
    BLK = 1024

    def body(h_ref, w_ref, b_ref, o_ref):
        hsum = h_ref[0] + h_ref[1]
        o_ref[...] = lax.dot_general(
            hsum, w_ref[...], (((1,), (1,)), ((), ())),
            preferred_element_type=jnp.float32) + b_ref[...]

    return pl.pallas_call(
        body,
        grid=(HP // BLK,),
        in_specs=[
            pl.BlockSpec((NC, BLK, D), lambda i: (0, i, 0)),
            pl.BlockSpec((D, D), lambda i: (0, 0)),
            pl.BlockSpec((1, D), lambda i: (0, 0)),
        ],
        out_specs=pl.BlockSpec((BLK, D), lambda i: (i, 0)),
        out_shape=jax.ShapeDtypeStruct((HP, D), jnp.float32),
    )(h2, W, b2)


def kernel(x, edge_index, W, b):
    src = edge_index[0].astype(jnp.int32)
    dst = edge_index[1].astype(jnp.int32)
    pad = E_PAD - N_EDGES
    src_p = jnp.concatenate(
        [src, jnp.zeros((pad,), jnp.int32)]).reshape(NW, KCH, CH)
    dst_pad = N_NODES + (jnp.arange(pad, dtype=jnp.int32) % (HP - N_NODES))
    dst_p = jnp.concatenate([dst, dst_pad]).reshape(NW, KCH, CH)
    h2 = _sc_segment_sum(x, src_p, dst_p)
    out = _tc_linear(h2, W, b.reshape(1, D))
    return out[:N_NODES]


# speedup vs baseline: 1.5240x; 1.5240x over previous
import functools

import jax
import jax.numpy as jnp
from jax import lax
from jax.experimental import pallas as pl
from jax.experimental.pallas import tpu as pltpu
from jax.experimental.pallas import tpu_sc as plsc

N_NODES = 10000
N_EDGES = 320000
D = 128

NC = 2
NS = 16
NW = NC * NS
CH = 128
KCH = 79
EPW = KCH * CH
E_PAD = NW * EPW
HP = 10240
ROWS_PER_TILE = HP // NS


def _sc_segment_sum(x, src2d, dst2d):
    mesh = plsc.VectorSubcoreMesh(core_axis_name="c", subcore_axis_name="s")

    @functools.partial(
        pl.kernel,
        mesh=mesh,
        out_type=jax.ShapeDtypeStruct((NC, HP, D), jnp.float32),
        scratch_types=[
            pltpu.VMEM((KCH, CH), jnp.int32),
            pltpu.VMEM((KCH, CH), jnp.int32),
            pltpu.VMEM((CH, D), jnp.float32),
            pltpu.VMEM_SHARED((HP, D), jnp.float32),
            pltpu.SemaphoreType.DMA,
        ],
    )
    def k(x_hbm, src_hbm, dst_hbm, out_hbm, src_v, dst_v, rows_v, h_sh, sem):
        c = lax.axis_index("c")
        s = lax.axis_index("s")
        wid = s * NC + c

        def zrow(i, _):
            def zcol(j, _):
                rows_v[i, pl.ds(j * 16, 16)] = jnp.zeros((16,), jnp.float32)
                return 0
            return lax.fori_loop(0, D // 16, zcol, 0)
        lax.fori_loop(0, CH, zrow, 0)

        base = s * ROWS_PER_TILE

        def zblk(t, _):
            pltpu.sync_copy(rows_v, h_sh.at[pl.ds(base + t * CH, CH)])
            return 0
        lax.fori_loop(0, ROWS_PER_TILE // CH, zblk, 0)
        plsc.subcore_barrier()

        pltpu.sync_copy(src_hbm.at[wid], src_v)
        pltpu.sync_copy(dst_hbm.at[wid], dst_v)

        def body(j, _):
            pltpu.async_copy(x_hbm.at[src_v.at[j]], rows_v, sem).wait()
            pltpu.sync_copy(rows_v, h_sh.at[dst_v.at[j]], add=True)
            return 0
        lax.fori_loop(0, KCH, body, 0)
        plsc.subcore_barrier()

        pltpu.sync_copy(h_sh.at[pl.ds(base, ROWS_PER_TILE)],
                        out_hbm.at[c, pl.ds(base, ROWS_PER_TILE)])

    return k(x, src2d, dst2d)


def _tc_linear(h2, W, b2):
    BLK = 1024

    def body(h_ref, w_ref, b_ref, o_ref):
        hsum = h_ref[0] + h_ref[1]
        o_ref[...] = lax.dot_general(
            hsum, w_ref[...], (((1,), (1,)), ((), ())),
            preferred_element_type=jnp.float32) + b_ref[...]

    return pl.pallas_call(
        body,
        grid=(HP // BLK,),
        in_specs=[
            pl.BlockSpec((NC, BLK, D), lambda i: (0, i, 0)),
            pl.BlockSpec((D, D), lambda i: (0, 0)),
            pl.BlockSpec((1, D), lambda i: (0, 0)),
        ],
        out_specs=pl.BlockSpec((BLK, D), lambda i: (i, 0)),
        out_shape=jax.ShapeDtypeStruct((HP, D), jnp.float32),
    )(h2, W, b2)


def kernel(x, edge_index, W, b):
    src = edge_index[0].astype(jnp.int32)
    dst = edge_index[1].astype(jnp.int32)
    pad = E_PAD - N_EDGES
    src_p = jnp.concatenate(
        [src, jnp.zeros((pad,), jnp.int32)]).reshape(NW, KCH, CH)
    dst_pad = N_NODES + (jnp.arange(pad, dtype=jnp.int32) % (HP - N_NODES))
    dst_p = jnp.concatenate([dst, dst_pad]).reshape(NW, KCH, CH)
    h2 = _sc_segment_sum(x, src_p, dst_p)
    out = _tc_linear(h2, W, b.reshape(1, D))
    return out[:N_NODES]
